# baseline (speedup 1.0000x reference)
import jax
import jax.numpy as jnp
from jax import lax
from jax.experimental import pallas as pl
from jax.experimental.pallas import tpu as pltpu
from jax.experimental.pallas import tpu_sc as plsc

N_NODES = 10000
D = 128
DH = D // 2
N_EDGES = 320000

NC = 2
NS = 16
CHUNK = 128
EDGES_PER_T = N_EDGES // NS
FULL_CHUNKS = EDGES_PER_T // CHUNK
TAIL = EDGES_PER_T - FULL_CHUNKS * CHUNK
AGG_ROWS = 10240
ZROWS = AGG_ROWS // NS
NBUF = 6
GLA = 5


def _seg_sum_body(xs_hbm, src_hbm, dst_hbm, zeros_hbm, out_hbm,
                  src_v, dst_v, rows, agg, gsem, ssem):
    c = lax.axis_index("c")
    s = lax.axis_index("s")
    x_hbm = xs_hbm.at[c]

    pltpu.sync_copy(src_hbm.at[s], src_v)
    pltpu.sync_copy(dst_hbm.at[s], dst_v)

    pltpu.sync_copy(zeros_hbm, rows[0])
    for k in range(ZROWS // CHUNK):
        pltpu.sync_copy(rows[0], agg.at[pl.ds(s * ZROWS + k * CHUNK, CHUNK)])
    plsc.subcore_barrier()

    def _idx(v, j):
        return v.at[pl.ds(j * CHUNK, CHUNK)]

    def _gather(j, b):
        pltpu.make_async_copy(x_hbm.at[_idx(src_v, j)], rows[b],
                              gsem[b]).start()

    def _drain(sem, b):
        pltpu.make_async_copy(zeros_hbm, rows[b], sem).wait()

    for b in range(GLA):
        _gather(b, b)

    def body(g, carry):
        for i in range(NBUF):
            j = g * NBUF + i
            _drain(gsem[i], i)
            pltpu.async_copy(rows[i], agg.at[_idx(dst_v, j)], ssem[i],
                             add=True)

            bn = (i + GLA) % NBUF

            @pl.when(j + GLA < FULL_CHUNKS)
            def _():
                @pl.when(j >= NBUF - GLA)
                def _():
                    _drain(ssem[bn], bn)
                _gather(j + GLA, bn)

        return carry

    lax.fori_loop(0, FULL_CHUNKS // NBUF, body, 0)

    for b in range(NBUF):
        _drain(ssem[b], b)

    toff = FULL_CHUNKS * CHUNK
    pltpu.make_async_copy(x_hbm.at[src_v.at[pl.ds(toff, TAIL)]],
                          rows[0].at[pl.ds(0, TAIL)], gsem[0]).start()
    pltpu.make_async_copy(x_hbm.at[src_v.at[pl.ds(toff, TAIL)]],
                          rows[0].at[pl.ds(0, TAIL)], gsem[0]).wait()
    pltpu.sync_copy(rows[0].at[pl.ds(0, TAIL)],
                    agg.at[dst_v.at[pl.ds(toff, TAIL)]], add=True)
    plsc.subcore_barrier()

    pltpu.sync_copy(agg.at[pl.ds(s * ZROWS, ZROWS)],
                    out_hbm.at[c].at[pl.ds(s * ZROWS, ZROWS)])


def _seg_sum(xs, src_r, dst_r, zeros):
    f = pl.kernel(
        _seg_sum_body,
        out_type=jax.ShapeDtypeStruct((NC, AGG_ROWS, DH), jnp.float32),
        mesh=plsc.VectorSubcoreMesh(core_axis_name="c", subcore_axis_name="s"),
        compiler_params=pltpu.CompilerParams(use_tc_tiling_on_sc=False),
        scratch_types=[
            pltpu.VMEM((EDGES_PER_T,), jnp.int32),
            pltpu.VMEM((EDGES_PER_T,), jnp.int32),
            [pltpu.VMEM((CHUNK, DH), jnp.float32) for _ in range(NBUF)],
            pltpu.VMEM_SHARED((AGG_ROWS, DH), jnp.float32),
            [pltpu.SemaphoreType.DMA for _ in range(NBUF)],
            [pltpu.SemaphoreType.DMA for _ in range(NBUF)],
        ],
    )
    return f(xs, src_r, dst_r, zeros)


BR = 1000


def _cat(a_ref):
    return jnp.concatenate([a_ref[0], a_ref[1]], axis=1)


def _tc1_body(scale_ref, x_ref, a_ref, w1_ref, b1_ref, w2_ref, b2_ref, o_ref):
    h = x_ref[...] * scale_ref[0, 0] + _cat(a_ref)
    t = jnp.maximum(
        jnp.dot(h, w1_ref[...], preferred_element_type=jnp.float32)
        + b1_ref[...], 0.0)
    t = jnp.dot(t, w2_ref[...], preferred_element_type=jnp.float32) + b2_ref[...]
    t = jnp.maximum(t, 0.0)
    o_ref[0] = t[:, :DH]
    o_ref[1] = t[:, DH:]


def _tc2_body(scale_ref, x_ref, a_ref, w1_ref, b1_ref, w2_ref, b2_ref,
              wm1_ref, bm1_ref, wm2_ref, bm2_ref, o_ref):
    h = _cat(x_ref) * scale_ref[0, 0] + _cat(a_ref)
    t = jnp.maximum(
        jnp.dot(h, w1_ref[...], preferred_element_type=jnp.float32)
        + b1_ref[...], 0.0)
    t = jnp.dot(t, w2_ref[...], preferred_element_type=jnp.float32) + b2_ref[...]
    t = jnp.maximum(t, 0.0)
    t = jnp.maximum(
        jnp.dot(t, wm1_ref[...], preferred_element_type=jnp.float32)
        + bm1_ref[...], 0.0)
    o_ref[...] = (jnp.dot(t, wm2_ref[...], preferred_element_type=jnp.float32)
                  + bm2_ref[...])


def _row_spec():
    return pl.BlockSpec((BR, D), lambda i: (i, 0))


def _split_spec():
    return pl.BlockSpec((NC, BR, DH), lambda i: (0, i, 0))


def _full_spec(shape):
    return pl.BlockSpec(shape, lambda i: tuple(0 for _ in shape))


def _tc_stage(body, scale, x, x_spec, agg, weights, out_shape, out_spec):
    in_specs = ([_full_spec((1, 1)), x_spec, _split_spec()]
                + [_full_spec(w.shape) for w in weights])
    return pl.pallas_call(
        body,
        grid=(N_NODES // BR,),
        in_specs=in_specs,
        out_specs=out_spec,
        out_shape=out_shape,
    )(scale, x, agg, *weights)


def kernel(x, edge_index, eps1, W11, b11, W12, b12, eps2, W21, b21, W22, b22,
           Wm1, bm1, Wm2, bm2):
    ei = edge_index.astype(jnp.int32)
    src_r = ei[0].reshape(NS, EDGES_PER_T)
    dst_r = ei[1].reshape(NS, EDGES_PER_T)
    zeros = jnp.zeros((CHUNK, DH), jnp.float32)

    scale1 = jnp.reshape(1.0 + eps1, (1, 1)).astype(jnp.float32)
    scale2 = jnp.reshape(1.0 + eps2, (1, 1)).astype(jnp.float32)
    b11r, b12r = b11.reshape(1, D), b12.reshape(1, D)
    b21r, b22r = b21.reshape(1, D), b22.reshape(1, D)
    bm1r, bm2r = bm1.reshape(1, D), bm2.reshape(1, D)

    xs = jnp.stack([x[:, :DH], x[:, DH:]])
    agg1 = _seg_sum(xs, src_r, dst_r, zeros)
    h1s = _tc_stage(_tc1_body, scale1, x, _row_spec(), agg1,
                    [W11, b11r, W12, b12r],
                    jax.ShapeDtypeStruct((NC, N_NODES, DH), jnp.float32),
                    _split_spec())
    agg2 = _seg_sum(h1s, src_r, dst_r, zeros)
    out = _tc_stage(_tc2_body, scale2, h1s, _split_spec(), agg2,
                    [W21, b21r, W22, b22r, Wm1, bm1r, Wm2, bm2r],
                    jax.ShapeDtypeStruct((N_NODES, D), jnp.float32),
                    _row_spec())
    return out


# speedup vs baseline: 1.0228x; 1.0228x over previous
import jax
import jax.numpy as jnp
from jax import lax
from jax.experimental import pallas as pl
from jax.experimental.pallas import tpu as pltpu
from jax.experimental.pallas import tpu_sc as plsc

N_NODES = 10000
D = 128
DH = D // 2
N_EDGES = 320000

NC = 2
NS = 16
CHUNK = 128
EDGES_PER_T = N_EDGES // NS
FULL_CHUNKS = EDGES_PER_T // CHUNK
TAIL = EDGES_PER_T - FULL_CHUNKS * CHUNK
AGG_ROWS = 10240
ZROWS = AGG_ROWS // NS
NBUF = 6
GLA = 4


def _seg_sum_body(xs_hbm, src_hbm, dst_hbm, zeros_hbm, out_hbm,
                  src_v, dst_v, rows, agg, gsem, ssem):
    c = lax.axis_index("c")
    s = lax.axis_index("s")
    x_hbm = xs_hbm.at[c]

    pltpu.sync_copy(src_hbm.at[s], src_v)
    pltpu.sync_copy(dst_hbm.at[s], dst_v)

    pltpu.sync_copy(zeros_hbm, rows[0])
    for k in range(ZROWS // CHUNK):
        pltpu.sync_copy(rows[0], agg.at[pl.ds(s * ZROWS + k * CHUNK, CHUNK)])
    plsc.subcore_barrier()

    def _idx(v, j):
        return v.at[pl.ds(j * CHUNK, CHUNK)]

    def _gather(j, b):
        pltpu.make_async_copy(x_hbm.at[_idx(src_v, j)], rows[b],
                              gsem[b]).start()

    def _drain(sem, b):
        pltpu.make_async_copy(zeros_hbm, rows[b], sem).wait()

    for b in range(GLA):
        _gather(b, b)

    def body(g, carry):
        for i in range(NBUF):
            j = g * NBUF + i
            _drain(gsem[i], i)
            pltpu.async_copy(rows[i], agg.at[_idx(dst_v, j)], ssem[i],
                             add=True)

            bn = (i + GLA) % NBUF

            @pl.when(j + GLA < FULL_CHUNKS)
            def _():
                @pl.when(j >= NBUF - GLA)
                def _():
                    _drain(ssem[bn], bn)
                _gather(j + GLA, bn)

        return carry

    lax.fori_loop(0, FULL_CHUNKS // NBUF, body, 0)

    for b in range(NBUF):
        _drain(ssem[b], b)

    toff = FULL_CHUNKS * CHUNK
    pltpu.make_async_copy(x_hbm.at[src_v.at[pl.ds(toff, TAIL)]],
                          rows[0].at[pl.ds(0, TAIL)], gsem[0]).start()
    pltpu.make_async_copy(x_hbm.at[src_v.at[pl.ds(toff, TAIL)]],
                          rows[0].at[pl.ds(0, TAIL)], gsem[0]).wait()
    pltpu.sync_copy(rows[0].at[pl.ds(0, TAIL)],
                    agg.at[dst_v.at[pl.ds(toff, TAIL)]], add=True)
    plsc.subcore_barrier()

    pltpu.sync_copy(agg.at[pl.ds(s * ZROWS, ZROWS)],
                    out_hbm.at[c].at[pl.ds(s * ZROWS, ZROWS)])


def _seg_sum(xs, src_r, dst_r, zeros):
    f = pl.kernel(
        _seg_sum_body,
        out_type=jax.ShapeDtypeStruct((NC, AGG_ROWS, DH), jnp.float32),
        mesh=plsc.VectorSubcoreMesh(core_axis_name="c", subcore_axis_name="s"),
        compiler_params=pltpu.CompilerParams(use_tc_tiling_on_sc=False),
        scratch_types=[
            pltpu.VMEM((EDGES_PER_T,), jnp.int32),
            pltpu.VMEM((EDGES_PER_T,), jnp.int32),
            [pltpu.VMEM((CHUNK, DH), jnp.float32) for _ in range(NBUF)],
            pltpu.VMEM_SHARED((AGG_ROWS, DH), jnp.float32),
            [pltpu.SemaphoreType.DMA for _ in range(NBUF)],
            [pltpu.SemaphoreType.DMA for _ in range(NBUF)],
        ],
    )
    return f(xs, src_r, dst_r, zeros)


BR = 2000


def _cat(a_ref):
    return jnp.concatenate([a_ref[0], a_ref[1]], axis=1)


def _tc1_body(scale_ref, x_ref, a_ref, w1_ref, b1_ref, w2_ref, b2_ref, o_ref):
    h = x_ref[...] * scale_ref[0, 0] + _cat(a_ref)
    t = jnp.maximum(
        jnp.dot(h, w1_ref[...], preferred_element_type=jnp.float32)
        + b1_ref[...], 0.0)
    t = jnp.dot(t, w2_ref[...], preferred_element_type=jnp.float32) + b2_ref[...]
    t = jnp.maximum(t, 0.0)
    o_ref[0] = t[:, :DH]
    o_ref[1] = t[:, DH:]


def _tc2_body(scale_ref, x_ref, a_ref, w1_ref, b1_ref, w2_ref, b2_ref,
              wm1_ref, bm1_ref, wm2_ref, bm2_ref, o_ref):
    h = _cat(x_ref) * scale_ref[0, 0] + _cat(a_ref)
    t = jnp.maximum(
        jnp.dot(h, w1_ref[...], preferred_element_type=jnp.float32)
        + b1_ref[...], 0.0)
    t = jnp.dot(t, w2_ref[...], preferred_element_type=jnp.float32) + b2_ref[...]
    t = jnp.maximum(t, 0.0)
    t = jnp.maximum(
        jnp.dot(t, wm1_ref[...], preferred_element_type=jnp.float32)
        + bm1_ref[...], 0.0)
    o_ref[...] = (jnp.dot(t, wm2_ref[...], preferred_element_type=jnp.float32)
                  + bm2_ref[...])


def _row_spec():
    return pl.BlockSpec((BR, D), lambda i: (i, 0))


def _split_spec():
    return pl.BlockSpec((NC, BR, DH), lambda i: (0, i, 0))


def _full_spec(shape):
    return pl.BlockSpec(shape, lambda i: tuple(0 for _ in shape))


def _tc_stage(body, scale, x, x_spec, agg, weights, out_shape, out_spec):
    in_specs = ([_full_spec((1, 1)), x_spec, _split_spec()]
                + [_full_spec(w.shape) for w in weights])
    return pl.pallas_call(
        body,
        grid=(N_NODES // BR,),
        in_specs=in_specs,
        out_specs=out_spec,
        out_shape=out_shape,
    )(scale, x, agg, *weights)


def kernel(x, edge_index, eps1, W11, b11, W12, b12, eps2, W21, b21, W22, b22,
           Wm1, bm1, Wm2, bm2):
    ei = edge_index.astype(jnp.int32)
    src_r = ei[0].reshape(NS, EDGES_PER_T)
    dst_r = ei[1].reshape(NS, EDGES_PER_T)
    zeros = jnp.zeros((CHUNK, DH), jnp.float32)

    scale1 = jnp.reshape(1.0 + eps1, (1, 1)).astype(jnp.float32)
    scale2 = jnp.reshape(1.0 + eps2, (1, 1)).astype(jnp.float32)
    b11r, b12r = b11.reshape(1, D), b12.reshape(1, D)
    b21r, b22r = b21.reshape(1, D), b22.reshape(1, D)
    bm1r, bm2r = bm1.reshape(1, D), bm2.reshape(1, D)

    xs = jnp.stack([x[:, :DH], x[:, DH:]])
    agg1 = _seg_sum(xs, src_r, dst_r, zeros)
    h1s = _tc_stage(_tc1_body, scale1, x, _row_spec(), agg1,
                    [W11, b11r, W12, b12r],
                    jax.ShapeDtypeStruct((NC, N_NODES, DH), jnp.float32),
                    _split_spec())
    agg2 = _seg_sum(h1s, src_r, dst_r, zeros)
    out = _tc_stage(_tc2_body, scale2, h1s, _split_spec(), agg2,
                    [W21, b21r, W22, b22r, Wm1, bm1r, Wm2, bm2r],
                    jax.ShapeDtypeStruct((N_NODES, D), jnp.float32),
                    _row_spec())
    return out
